# Initial kernel scaffold; baseline (speedup 1.0000x reference)
import jax
import jax.numpy as jnp
from jax.experimental import pallas as pl


def kernel(edge_index, edge_type, edge_weight, W0, wc0, W1, wc1, W2, wc2):
    raise NotImplementedError("write your pallas kernel here")


# speedup vs baseline: 9.7653x; 9.7653x over previous
import functools

import jax
import jax.numpy as jnp
from jax import lax
from jax.experimental import pallas as pl
from jax.experimental.pallas import tpu as pltpu
from jax.experimental.pallas import tpu_sc as plsc

_N = 10000
_E = 320000
_R = 8
_B = 4
_H = 128
_OUT = 16
_HH = _H // 2

_C = 128
_NCHUNK = 2560
_EPAD = _NCHUNK * _C
_NC = 2
_NS = 16
_NW = _NC * _NS
_NPAD = 10240
_RPS = _NPAD // _NS
_ZROWS = 128

_BLK = 2000


_BLK0 = 1000


def _combine0_body(wcb_ref, w_ref, out_ref):
    acc = w_ref[0][:, None, :] * wcb_ref[0][None, :, :]
    for b in range(1, _B):
        acc = acc + w_ref[b][:, None, :] * wcb_ref[b][None, :, :]
    out_ref[...] = acc


def _combine0(wc0, W0):
    wcb = jnp.broadcast_to(wc0.T[:, :, None], (_B, _R, _H))
    return pl.pallas_call(
        _combine0_body,
        grid=(_N // _BLK0,),
        in_specs=[
            pl.BlockSpec((_B, _R, _H), lambda n: (0, 0, 0)),
            pl.BlockSpec((_B, _BLK0, _H), lambda n: (0, n, 0)),
        ],
        out_specs=pl.BlockSpec((_BLK0, _R, _H), lambda n: (n, 0, 0)),
        out_shape=jax.ShapeDtypeStruct((_N, _R, _H), jnp.float32),
    )(wcb, W0)


def _combine_w_body(wcb_ref, w_ref, out_ref):
    acc = w_ref[0][:, None, :] * wcb_ref[0][None, :, :]
    for b in range(1, _B):
        acc = acc + w_ref[b][:, None, :] * wcb_ref[b][None, :, :]
    out_ref[...] = acc


def _combine_w(wc, W):
    wcb = jnp.broadcast_to(wc.T[:, :, None], (_B, _R, _H))
    out = pl.pallas_call(
        _combine_w_body,
        grid=(1,),
        in_specs=[
            pl.BlockSpec((_B, _R, _H), lambda i: (0, 0, 0)),
            pl.BlockSpec((_B, _H, _H), lambda i: (0, 0, 0)),
        ],
        out_specs=pl.BlockSpec((_H, _R, _H), lambda i: (0, 0, 0)),
        out_shape=jax.ShapeDtypeStruct((_H, _R, _H), jnp.float32),
    )(wcb, W)
    return out.reshape(_H * _R, _H)


def _transform_body(a0_ref, a1_ref, wf_ref, out_ref):
    h = jnp.maximum(a0_ref[...] + a1_ref[...], 0.0)
    out_ref[0] = jnp.dot(h, wf_ref[...], preferred_element_type=jnp.float32)


def _transform(a0, a1, wflat):
    return pl.pallas_call(
        _transform_body,
        grid=(_N // _BLK, _R),
        in_specs=[
            pl.BlockSpec((_BLK, _H), lambda n, r: (n, 0)),
            pl.BlockSpec((_BLK, _H), lambda n, r: (n, 0)),
            pl.BlockSpec((_H, _H), lambda n, r: (r, 0)),
        ],
        out_specs=pl.BlockSpec((1, _BLK, _H), lambda n, r: (r, n, 0)),
        out_shape=jax.ShapeDtypeStruct((_R, _N, _H), jnp.float32),
    )(a0, a1, wflat)


def _softmax_body(b0_ref, b1_ref, out_ref):
    x = b0_ref[...] + b1_ref[...]
    m = jnp.max(x, axis=1, keepdims=True)
    e = jnp.exp(x - m)
    out_ref[...] = e / jnp.sum(e, axis=1, keepdims=True)


def _softmax(b0, b1):
    return pl.pallas_call(
        _softmax_body,
        grid=(_N // _BLK,),
        in_specs=[
            pl.BlockSpec((_BLK, _OUT), lambda n: (n, 0)),
            pl.BlockSpec((_BLK, _OUT), lambda n: (n, 0)),
        ],
        out_specs=pl.BlockSpec((_BLK, _OUT), lambda n: (n, 0)),
        out_shape=jax.ShapeDtypeStruct((_N, _OUT), jnp.float32),
    )(b0, b1)


_CPW = _NCHUNK // _NW
_ZR = 64


def _make_sc_agg():
    mesh = plsc.VectorSubcoreMesh(core_axis_name="c", subcore_axis_name="s")

    @functools.partial(
        pl.kernel,
        mesh=mesh,
        out_type=jax.ShapeDtypeStruct((_NC, _NPAD, _H), jnp.float32),
        scratch_types=[
            pltpu.VMEM((_C * 16,), jnp.float32),
            pltpu.VMEM((_C,), jnp.int32),
            pltpu.VMEM((_C,), jnp.int32),
            pltpu.VMEM((_C, _H), jnp.float32),
            pltpu.VMEM((_ZR, _H), jnp.float32),
            pltpu.VMEM_SHARED((_NPAD, _H), jnp.float32),
            pltpu.SemaphoreType.DMA,
        ],
    )
    def agg(gidx_hbm, dst_hbm, wb_hbm, table_hbm, out_hbm,
            wb_v, gidx_v, sidx_v, rows_v, zbuf_v, acc, sem):
        cid = lax.axis_index("c")
        sid = lax.axis_index("s")
        base = (cid * _NS + sid) * _CPW

        def zloop(i, carry):
            for j in range(_H // 16):
                zbuf_v[i, pl.ds(j * 16, 16)] = jnp.zeros((16,), jnp.float32)
            return carry
        lax.fori_loop(0, _ZR, zloop, 0)
        for i in range(_RPS // _ZR):
            pltpu.sync_copy(
                zbuf_v, acc.at[pl.ds(sid * _RPS + i * _ZR, _ZR)])

        plsc.subcore_barrier()

        def chunk_body(t, carry):
            row = base + t
            pltpu.sync_copy(gidx_hbm.at[row], gidx_v)
            pltpu.sync_copy(dst_hbm.at[row], sidx_v)
            pltpu.sync_copy(wb_hbm.at[row], wb_v)
            pltpu.async_copy(table_hbm.at[gidx_v], rows_v, sem).wait()

            for i in range(_C):
                wv = wb_v[pl.ds(i * 16, 16)]
                for j in range(_H // 16):
                    sl = pl.ds(j * 16, 16)
                    rows_v[i, sl] = rows_v[i, sl] * wv

            pltpu.sync_copy(rows_v, acc.at[sidx_v], add=True)
            return carry
        lax.fori_loop(0, _CPW, chunk_body, 0)

        plsc.subcore_barrier()

        pltpu.sync_copy(
            acc.at[pl.ds(sid * _RPS, _RPS)],
            out_hbm.at[cid, pl.ds(sid * _RPS, _RPS)])

    return agg


def _pad2d(x, dtype):
    pad = jnp.zeros((_EPAD - _E,), dtype)
    return jnp.concatenate([x.astype(dtype), pad]).reshape(_NCHUNK, _C)


def kernel(edge_index, edge_type, edge_weight, W0, wc0, W1, wc1, W2, wc2):
    gidx2 = _pad2d(edge_type.astype(jnp.int32) * _N + edge_index[0],
                   jnp.int32)
    dst2 = _pad2d(edge_index[1], jnp.int32)
    wb2 = jnp.broadcast_to(
        _pad2d(edge_weight, jnp.float32).reshape(_EPAD, 1),
        (_EPAD, 16)).reshape(_NCHUNK, _C * 16)

    agg = _make_sc_agg()

    W2p = jnp.pad(W2, ((0, 0), (0, 0), (0, _H - _OUT)))

    t0 = _combine0(wc0, W0).reshape(_N * _R, _H)
    a = agg(gidx2, dst2, wb2, t0)
    t1 = _transform(a[0, :_N], a[1, :_N], _combine_w(wc1, W1)).reshape(
        _R * _N, _H)
    a = agg(gidx2, dst2, wb2, t1)
    t2 = _transform(a[0, :_N], a[1, :_N], _combine_w(wc2, W2p)).reshape(
        _R * _N, _H)
    b = agg(gidx2, dst2, wb2, t2)
    return _softmax(b[0, :_N, :_OUT], b[1, :_N, :_OUT])


# speedup vs baseline: 13.0176x; 1.3330x over previous
import functools

import jax
import jax.numpy as jnp
from jax import lax
from jax.experimental import pallas as pl
from jax.experimental.pallas import tpu as pltpu
from jax.experimental.pallas import tpu_sc as plsc

_N = 10000
_E = 320000
_R = 8
_B = 4
_H = 128
_OUT = 16
_HH = _H // 2

_C = 128
_NCHUNK = 2560
_EPAD = _NCHUNK * _C
_NC = 2
_NS = 16
_NW = _NC * _NS
_NPAD = 10240
_RPS = _NPAD // _NS
_ZROWS = 128

_BLK = 2000


_BLK0 = 1000


def _combine0_body(wcb_ref, w_ref, out_ref):
    acc = w_ref[0][:, None, :] * wcb_ref[0][None, :, :]
    for b in range(1, _B):
        acc = acc + w_ref[b][:, None, :] * wcb_ref[b][None, :, :]
    out_ref[...] = acc


def _combine0(wc0, W0):
    wcb = jnp.broadcast_to(wc0.T[:, :, None], (_B, _R, _H))
    return pl.pallas_call(
        _combine0_body,
        grid=(_N // _BLK0,),
        in_specs=[
            pl.BlockSpec((_B, _R, _H), lambda n: (0, 0, 0)),
            pl.BlockSpec((_B, _BLK0, _H), lambda n: (0, n, 0)),
        ],
        out_specs=pl.BlockSpec((_BLK0, _R, _H), lambda n: (n, 0, 0)),
        out_shape=jax.ShapeDtypeStruct((_N, _R, _H), jnp.float32),
    )(wcb, W0)


def _combine_w_body(wcb_ref, w_ref, out_ref):
    acc = w_ref[0][:, None, :] * wcb_ref[0][None, :, :]
    for b in range(1, _B):
        acc = acc + w_ref[b][:, None, :] * wcb_ref[b][None, :, :]
    out_ref[...] = acc


def _combine_w(wc, W):
    wcb = jnp.broadcast_to(wc.T[:, :, None], (_B, _R, _H))
    out = pl.pallas_call(
        _combine_w_body,
        grid=(1,),
        in_specs=[
            pl.BlockSpec((_B, _R, _H), lambda i: (0, 0, 0)),
            pl.BlockSpec((_B, _H, _H), lambda i: (0, 0, 0)),
        ],
        out_specs=pl.BlockSpec((_H, _R, _H), lambda i: (0, 0, 0)),
        out_shape=jax.ShapeDtypeStruct((_H, _R, _H), jnp.float32),
    )(wcb, W)
    return out.reshape(_H * _R, _H)


def _transform_body(a0_ref, a1_ref, wf_ref, out_ref):
    h = jnp.maximum(a0_ref[...] + a1_ref[...], 0.0)
    out_ref[0] = jnp.dot(h, wf_ref[...], preferred_element_type=jnp.float32)


def _transform(a0, a1, wflat):
    return pl.pallas_call(
        _transform_body,
        grid=(_N // _BLK, _R),
        in_specs=[
            pl.BlockSpec((_BLK, _H), lambda n, r: (n, 0)),
            pl.BlockSpec((_BLK, _H), lambda n, r: (n, 0)),
            pl.BlockSpec((_H, _H), lambda n, r: (r, 0)),
        ],
        out_specs=pl.BlockSpec((1, _BLK, _H), lambda n, r: (r, n, 0)),
        out_shape=jax.ShapeDtypeStruct((_R, _N, _H), jnp.float32),
    )(a0, a1, wflat)


def _softmax_body(b0_ref, b1_ref, out_ref):
    x = b0_ref[...] + b1_ref[...]
    m = jnp.max(x, axis=1, keepdims=True)
    e = jnp.exp(x - m)
    out_ref[...] = e / jnp.sum(e, axis=1, keepdims=True)


def _softmax(b0, b1):
    return pl.pallas_call(
        _softmax_body,
        grid=(_N // _BLK,),
        in_specs=[
            pl.BlockSpec((_BLK, _OUT), lambda n: (n, 0)),
            pl.BlockSpec((_BLK, _OUT), lambda n: (n, 0)),
        ],
        out_specs=pl.BlockSpec((_BLK, _OUT), lambda n: (n, 0)),
        out_shape=jax.ShapeDtypeStruct((_N, _OUT), jnp.float32),
    )(b0, b1)


_CPW = _NCHUNK // _NW
_ZR = 64


def _make_sc_agg():
    mesh = plsc.VectorSubcoreMesh(core_axis_name="c", subcore_axis_name="s")

    @functools.partial(
        pl.kernel,
        mesh=mesh,
        out_type=jax.ShapeDtypeStruct((_NC, _NPAD, _H), jnp.float32),
        scratch_types=[
            pltpu.VMEM((_C * 16,), jnp.float32),
            pltpu.VMEM((_C,), jnp.int32),
            pltpu.VMEM((_C,), jnp.int32),
            pltpu.VMEM((_C, _H), jnp.float32),
            pltpu.VMEM((_C * 16,), jnp.float32),
            pltpu.VMEM((_C,), jnp.int32),
            pltpu.VMEM((_C,), jnp.int32),
            pltpu.VMEM((_C, _H), jnp.float32),
            pltpu.VMEM((_ZR, _H), jnp.float32),
            pltpu.VMEM_SHARED((_NPAD, _H), jnp.float32),
            pltpu.SemaphoreType.DMA,
            pltpu.SemaphoreType.DMA,
        ],
    )
    def agg(gidx_hbm, dst_hbm, wb_hbm, table_hbm, out_hbm,
            wb0, gidx0, sidx0, rows0, wb1, gidx1, sidx1, rows1,
            zbuf_v, acc, sem0, sem1):
        cid = lax.axis_index("c")
        sid = lax.axis_index("s")
        base = (cid * _NS + sid) * _CPW

        def zloop(i, carry):
            for j in range(_H // 16):
                zbuf_v[i, pl.ds(j * 16, 16)] = jnp.zeros((16,), jnp.float32)
            return carry
        lax.fori_loop(0, _ZR, zloop, 0)
        for i in range(_RPS // _ZR):
            pltpu.sync_copy(
                zbuf_v, acc.at[pl.ds(sid * _RPS + i * _ZR, _ZR)])

        plsc.subcore_barrier()

        def stage(row, gidx_b, sidx_b, wb_b):
            pltpu.sync_copy(gidx_hbm.at[row], gidx_b)
            pltpu.sync_copy(dst_hbm.at[row], sidx_b)
            pltpu.sync_copy(wb_hbm.at[row], wb_b)

        def scale_scatter(rows_b, wb_b, sidx_b):
            for i in range(_C):
                wv = wb_b[pl.ds(i * 16, 16)]
                for j in range(_H // 16):
                    sl = pl.ds(j * 16, 16)
                    rows_b[i, sl] = rows_b[i, sl] * wv
            pltpu.sync_copy(rows_b, acc.at[sidx_b], add=True)

        stage(base, gidx0, sidx0, wb0)
        pltpu.async_copy(table_hbm.at[gidx0], rows0, sem0)

        def pair_body(tp, carry):
            t = base + tp * 2
            stage(t + 1, gidx1, sidx1, wb1)
            pltpu.async_copy(table_hbm.at[gidx1], rows1, sem1)
            pltpu.make_async_copy(table_hbm.at[gidx0], rows0, sem0).wait()
            scale_scatter(rows0, wb0, sidx0)
            row2 = jnp.minimum(t + 2, _NCHUNK - 1)
            stage(row2, gidx0, sidx0, wb0)
            pltpu.async_copy(table_hbm.at[gidx0], rows0, sem0)
            pltpu.make_async_copy(table_hbm.at[gidx1], rows1, sem1).wait()
            scale_scatter(rows1, wb1, sidx1)
            return carry
        lax.fori_loop(0, _CPW // 2, pair_body, 0)
        pltpu.make_async_copy(table_hbm.at[gidx0], rows0, sem0).wait()

        plsc.subcore_barrier()

        pltpu.sync_copy(
            acc.at[pl.ds(sid * _RPS, _RPS)],
            out_hbm.at[cid, pl.ds(sid * _RPS, _RPS)])

    return agg


def _pad2d(x, dtype):
    pad = jnp.zeros((_EPAD - _E,), dtype)
    return jnp.concatenate([x.astype(dtype), pad]).reshape(_NCHUNK, _C)


def kernel(edge_index, edge_type, edge_weight, W0, wc0, W1, wc1, W2, wc2):
    gidx2 = _pad2d(edge_type.astype(jnp.int32) * _N + edge_index[0],
                   jnp.int32)
    dst2 = _pad2d(edge_index[1], jnp.int32)
    wb2 = jnp.broadcast_to(
        _pad2d(edge_weight, jnp.float32).reshape(_EPAD, 1),
        (_EPAD, 16)).reshape(_NCHUNK, _C * 16)

    agg = _make_sc_agg()

    W2p = jnp.pad(W2, ((0, 0), (0, 0), (0, _H - _OUT)))

    t0 = _combine0(wc0, W0).reshape(_N * _R, _H)
    a = agg(gidx2, dst2, wb2, t0)
    t1 = _transform(a[0, :_N], a[1, :_N], _combine_w(wc1, W1)).reshape(
        _R * _N, _H)
    a = agg(gidx2, dst2, wb2, t1)
    t2 = _transform(a[0, :_N], a[1, :_N], _combine_w(wc2, W2p)).reshape(
        _R * _N, _H)
    b = agg(gidx2, dst2, wb2, t2)
    return _softmax(b[0, :_N, :_OUT], b[1, :_N, :_OUT])


# speedup vs baseline: 13.4340x; 1.0320x over previous
import functools

import jax
import jax.numpy as jnp
from jax import lax
from jax.experimental import pallas as pl
from jax.experimental.pallas import tpu as pltpu
from jax.experimental.pallas import tpu_sc as plsc

_N = 10000
_E = 320000
_R = 8
_B = 4
_H = 128
_OUT = 16
_HH = _H // 2

_C = 128
_NCHUNK = 2560
_EPAD = _NCHUNK * _C
_NC = 2
_NS = 16
_NW = _NC * _NS
_NPAD = 10240
_RPS = _NPAD // _NS
_ZROWS = 128

_BLK = 2000


_BLK0 = 1000


def _combine0_body(wcb_ref, w_ref, out_ref):
    acc = w_ref[0][:, None, :] * wcb_ref[0][None, :, :]
    for b in range(1, _B):
        acc = acc + w_ref[b][:, None, :] * wcb_ref[b][None, :, :]
    out_ref[...] = acc


def _combine0(wc0, W0):
    wcb = jnp.broadcast_to(wc0.T[:, :, None], (_B, _R, _H))
    return pl.pallas_call(
        _combine0_body,
        grid=(_N // _BLK0,),
        in_specs=[
            pl.BlockSpec((_B, _R, _H), lambda n: (0, 0, 0)),
            pl.BlockSpec((_B, _BLK0, _H), lambda n: (0, n, 0)),
        ],
        out_specs=pl.BlockSpec((_BLK0, _R, _H), lambda n: (n, 0, 0)),
        out_shape=jax.ShapeDtypeStruct((_N, _R, _H), jnp.float32),
    )(wcb, W0)


def _combine_w_body(wcb_ref, w_ref, out_ref):
    acc = w_ref[0][:, None, :] * wcb_ref[0][None, :, :]
    for b in range(1, _B):
        acc = acc + w_ref[b][:, None, :] * wcb_ref[b][None, :, :]
    out_ref[...] = acc


def _combine_w(wc, W):
    wcb = jnp.broadcast_to(wc.T[:, :, None], (_B, _R, _H))
    out = pl.pallas_call(
        _combine_w_body,
        grid=(1,),
        in_specs=[
            pl.BlockSpec((_B, _R, _H), lambda i: (0, 0, 0)),
            pl.BlockSpec((_B, _H, _H), lambda i: (0, 0, 0)),
        ],
        out_specs=pl.BlockSpec((_H, _R, _H), lambda i: (0, 0, 0)),
        out_shape=jax.ShapeDtypeStruct((_H, _R, _H), jnp.float32),
    )(wcb, W)
    return out.reshape(_H * _R, _H)


def _transform_body(a0_ref, a1_ref, wf_ref, out_ref):
    h = jnp.maximum(a0_ref[...] + a1_ref[...], 0.0)
    out_ref[0] = jnp.dot(h, wf_ref[...], preferred_element_type=jnp.float32)


def _transform(a0, a1, wflat):
    return pl.pallas_call(
        _transform_body,
        grid=(_N // _BLK, _R),
        in_specs=[
            pl.BlockSpec((_BLK, _H), lambda n, r: (n, 0)),
            pl.BlockSpec((_BLK, _H), lambda n, r: (n, 0)),
            pl.BlockSpec((_H, _H), lambda n, r: (r, 0)),
        ],
        out_specs=pl.BlockSpec((1, _BLK, _H), lambda n, r: (r, n, 0)),
        out_shape=jax.ShapeDtypeStruct((_R, _N, _H), jnp.float32),
    )(a0, a1, wflat)


def _softmax_body(b0_ref, b1_ref, out_ref):
    x = b0_ref[...] + b1_ref[...]
    m = jnp.max(x, axis=1, keepdims=True)
    e = jnp.exp(x - m)
    out_ref[...] = e / jnp.sum(e, axis=1, keepdims=True)


def _softmax(b0, b1):
    return pl.pallas_call(
        _softmax_body,
        grid=(_N // _BLK,),
        in_specs=[
            pl.BlockSpec((_BLK, _OUT), lambda n: (n, 0)),
            pl.BlockSpec((_BLK, _OUT), lambda n: (n, 0)),
        ],
        out_specs=pl.BlockSpec((_BLK, _OUT), lambda n: (n, 0)),
        out_shape=jax.ShapeDtypeStruct((_N, _OUT), jnp.float32),
    )(b0, b1)


_CPW = _NCHUNK // _NW
_ZR = 64


def _make_sc_agg():
    mesh = plsc.VectorSubcoreMesh(core_axis_name="c", subcore_axis_name="s")

    @functools.partial(
        pl.kernel,
        mesh=mesh,
        out_type=jax.ShapeDtypeStruct((_NC, _NPAD, _H), jnp.float32),
        scratch_types=[
            pltpu.VMEM((_C * 16,), jnp.float32),
            pltpu.VMEM((_C,), jnp.int32),
            pltpu.VMEM((_C,), jnp.int32),
            pltpu.VMEM((_C, _H), jnp.float32),
            pltpu.VMEM((_C * 16,), jnp.float32),
            pltpu.VMEM((_C,), jnp.int32),
            pltpu.VMEM((_C,), jnp.int32),
            pltpu.VMEM((_C, _H), jnp.float32),
            pltpu.VMEM((_ZR, _H), jnp.float32),
            pltpu.VMEM_SHARED((_NPAD, _H), jnp.float32),
            pltpu.SemaphoreType.DMA,
            pltpu.SemaphoreType.DMA,
            pltpu.SemaphoreType.DMA,
            pltpu.SemaphoreType.DMA,
        ],
    )
    def agg(gidx_hbm, dst_hbm, wb_hbm, table_hbm, out_hbm,
            wb0, gidx0, sidx0, rows0, wb1, gidx1, sidx1, rows1,
            zbuf_v, acc, sem0, sem1, semA0, semA1):
        cid = lax.axis_index("c")
        sid = lax.axis_index("s")
        base = (cid * _NS + sid) * _CPW

        def zloop(i, carry):
            for j in range(_H // 16):
                zbuf_v[i, pl.ds(j * 16, 16)] = jnp.zeros((16,), jnp.float32)
            return carry
        lax.fori_loop(0, _ZR, zloop, 0)
        for i in range(_RPS // _ZR):
            pltpu.sync_copy(
                zbuf_v, acc.at[pl.ds(sid * _RPS + i * _ZR, _ZR)])

        plsc.subcore_barrier()

        def stage(row, gidx_b, sidx_b, wb_b, semA):
            pltpu.async_copy(gidx_hbm.at[row], gidx_b, semA)
            pltpu.async_copy(dst_hbm.at[row], sidx_b, semA)
            pltpu.async_copy(wb_hbm.at[row], wb_b, semA)

        def stage_wait(gidx_b, sidx_b, wb_b, semA):
            pltpu.make_async_copy(gidx_hbm.at[0], gidx_b, semA).wait()
            pltpu.make_async_copy(dst_hbm.at[0], sidx_b, semA).wait()
            pltpu.make_async_copy(wb_hbm.at[0], wb_b, semA).wait()

        def scale_scatter(rows_b, wb_b, sidx_b):
            for i in range(_C):
                wv = wb_b[pl.ds(i * 16, 16)]
                for j in range(_H // 16):
                    sl = pl.ds(j * 16, 16)
                    rows_b[i, sl] = rows_b[i, sl] * wv
            pltpu.sync_copy(rows_b, acc.at[sidx_b], add=True)

        stage(base, gidx0, sidx0, wb0, semA0)
        stage_wait(gidx0, sidx0, wb0, semA0)
        pltpu.async_copy(table_hbm.at[gidx0], rows0, sem0)
        stage(base + 1, gidx1, sidx1, wb1, semA1)

        def pair_body(tp, carry):
            t = base + tp * 2
            stage_wait(gidx1, sidx1, wb1, semA1)
            pltpu.async_copy(table_hbm.at[gidx1], rows1, sem1)
            pltpu.make_async_copy(table_hbm.at[gidx0], rows0, sem0).wait()
            scale_scatter(rows0, wb0, sidx0)
            stage(jnp.minimum(t + 2, _NCHUNK - 1), gidx0, sidx0, wb0, semA0)
            stage_wait(gidx0, sidx0, wb0, semA0)
            pltpu.async_copy(table_hbm.at[gidx0], rows0, sem0)
            pltpu.make_async_copy(table_hbm.at[gidx1], rows1, sem1).wait()
            scale_scatter(rows1, wb1, sidx1)
            stage(jnp.minimum(t + 3, _NCHUNK - 1), gidx1, sidx1, wb1, semA1)
            return carry
        lax.fori_loop(0, _CPW // 2, pair_body, 0)
        pltpu.make_async_copy(table_hbm.at[gidx0], rows0, sem0).wait()
        stage_wait(gidx1, sidx1, wb1, semA1)

        plsc.subcore_barrier()

        pltpu.sync_copy(
            acc.at[pl.ds(sid * _RPS, _RPS)],
            out_hbm.at[cid, pl.ds(sid * _RPS, _RPS)])

    return agg


def _pad2d(x, dtype):
    pad = jnp.zeros((_EPAD - _E,), dtype)
    return jnp.concatenate([x.astype(dtype), pad]).reshape(_NCHUNK, _C)


def kernel(edge_index, edge_type, edge_weight, W0, wc0, W1, wc1, W2, wc2):
    gidx2 = _pad2d(edge_type.astype(jnp.int32) * _N + edge_index[0],
                   jnp.int32)
    dst2 = _pad2d(edge_index[1], jnp.int32)
    wb2 = jnp.broadcast_to(
        _pad2d(edge_weight, jnp.float32).reshape(_EPAD, 1),
        (_EPAD, 16)).reshape(_NCHUNK, _C * 16)

    agg = _make_sc_agg()

    W2p = jnp.pad(W2, ((0, 0), (0, 0), (0, _H - _OUT)))

    t0 = _combine0(wc0, W0).reshape(_N * _R, _H)
    a = agg(gidx2, dst2, wb2, t0)
    t1 = _transform(a[0, :_N], a[1, :_N], _combine_w(wc1, W1)).reshape(
        _R * _N, _H)
    a = agg(gidx2, dst2, wb2, t1)
    t2 = _transform(a[0, :_N], a[1, :_N], _combine_w(wc2, W2p)).reshape(
        _R * _N, _H)
    b = agg(gidx2, dst2, wb2, t2)
    return _softmax(b[0, :_N, :_OUT], b[1, :_N, :_OUT])
